# Initial kernel scaffold; baseline (speedup 1.0000x reference)
import jax
import jax.numpy as jnp
from jax.experimental import pallas as pl


def kernel(x, edge_index, W_pre, b_pre, g_pre, be_pre, Wk, Wq, Wv, Ws, bk, bq, bv, bs, gamma, beta, Wh1, bh1, Wh2, bh2):
    raise NotImplementedError("write your pallas kernel here")


# speedup vs baseline: 4.6383x; 4.6383x over previous
import functools

import jax
import jax.numpy as jnp
from jax import lax
from jax.experimental import pallas as pl
from jax.experimental.pallas import tpu as pltpu
from jax.experimental.pallas import tpu_sc as plsc

N = 10000
E = 320000
D = 128
L = 4
D_OUT = 51
EPS = 1e-5

NC = 2
NS = 16
NW = NC * NS
EW = E // NW
B = 80
NCHUNK = EW // B
NPAD = 10240
RPS = NPAD // NS
ZROWS = 128


def _bn(h, g, b):
    mu = jnp.mean(h, axis=0, keepdims=True)
    hc = h - mu
    var = jnp.mean(hc * hc, axis=0, keepdims=True)
    return g * hc * lax.rsqrt(var + EPS) + b


def _post(a, g, b):
    o = jax.nn.relu(_bn(a, g, b))
    nrm = jnp.sqrt(jnp.sum(o * o, axis=-1, keepdims=True))
    return o / (nrm + 1e-12)


def _proj(h, w, b):
    return jnp.dot(h, w, preferred_element_type=jnp.float32) + b


def _pre_body(x, wpre, bpre, gpre, bepre, wk, wq, wv, ws, bk, bq, bv, bs,
              h_o, k_o, q_o, v_o, s_o):
    h = _proj(x[...], wpre[...], bpre[...])
    h = _post(h, gpre[...], bepre[...])
    h_o[...] = h
    k_o[...] = _proj(h, wk[...], bk[...])
    q_o[...] = _proj(h, wq[...], bq[...])
    v_o[...] = _proj(h, wv[...], bv[...])
    s_o[...] = _proj(h, ws[...], bs[...])


_pre_call = pl.pallas_call(
    _pre_body,
    out_shape=tuple(jax.ShapeDtypeStruct((N, D), jnp.float32) for _ in range(5)),
)


def _mid_body(aggr, h, s, g, be, wk, wq, wv, ws, bk, bq, bv, bs,
              h_o, k_o, q_o, v_o, s_o):
    a = aggr[0:N, :] + aggr[NPAD:NPAD + N, :] + s[...]
    h2 = h[...] + _post(a, g[...], be[...])
    h_o[...] = h2
    k_o[...] = _proj(h2, wk[...], bk[...])
    q_o[...] = _proj(h2, wq[...], bq[...])
    v_o[...] = _proj(h2, wv[...], bv[...])
    s_o[...] = _proj(h2, ws[...], bs[...])


_mid_call = pl.pallas_call(
    _mid_body,
    out_shape=tuple(jax.ShapeDtypeStruct((N, D), jnp.float32) for _ in range(5)),
)


def _final_body(aggr, h, s, g, be, wh1, bh1, wh2, bh2, pred_o):
    a = aggr[0:N, :] + aggr[NPAD:NPAD + N, :] + s[...]
    h2 = h[...] + _post(a, g[...], be[...])
    z = jax.nn.relu(_proj(h2, wh1[...], bh1[...]))
    pred_o[...] = _proj(z, wh2[...], bh2[...])


_final_call = pl.pallas_call(
    _final_body,
    out_shape=jax.ShapeDtypeStruct((N, D_OUT), jnp.float32),
)


_sc_mesh = plsc.VectorSubcoreMesh(core_axis_name="c", subcore_axis_name="s")


@functools.partial(
    pl.kernel,
    out_type=jax.ShapeDtypeStruct((2 * NPAD, D), jnp.float32),
    mesh=_sc_mesh,
    scratch_types=[
        pltpu.VMEM((B,), jnp.int32),
        pltpu.VMEM((B,), jnp.int32),
        pltpu.VMEM((B, D), jnp.float32),
        pltpu.VMEM((B, D), jnp.float32),
        pltpu.VMEM((B, D), jnp.float32),
        pltpu.VMEM((ZROWS, D), jnp.float32),
        pltpu.VMEM_SHARED((NPAD, D), jnp.float32),
        pltpu.SemaphoreType.DMA,
        pltpu.SemaphoreType.DMA,
        pltpu.SemaphoreType.DMA,
    ],
)
def _edge_call(k_hbm, q_hbm, v_hbm, src_hbm, dst_hbm, out_hbm,
               srcv, dstv, krows, qrows, vrows, zbuf, aggr,
               sem1, sem2, sem3):
    cid = lax.axis_index("c")
    sid = lax.axis_index("s")
    wid = sid * NC + cid

    zeros16 = jnp.zeros((16,), jnp.float32)

    def zrow(r, _):
        for t in range(D // 16):
            zbuf[r, pl.ds(t * 16, 16)] = zeros16
        return 0

    lax.fori_loop(0, ZROWS, zrow, 0)
    for t in range(RPS // ZROWS):
        pltpu.sync_copy(zbuf, aggr.at[pl.ds(sid * RPS + t * ZROWS, ZROWS)])
    plsc.subcore_barrier()

    ebase = wid * EW

    def chunk(j, _):
        e = ebase + j * B
        pltpu.sync_copy(src_hbm.at[pl.ds(e, B)], srcv)
        pltpu.sync_copy(dst_hbm.at[pl.ds(e, B)], dstv)
        cp1 = pltpu.async_copy(k_hbm.at[dstv], krows, sem1)
        cp2 = pltpu.async_copy(q_hbm.at[srcv], qrows, sem2)
        cp3 = pltpu.async_copy(v_hbm.at[srcv], vrows, sem3)
        cp1.wait()
        cp2.wait()
        cp3.wait()

        def row(r, _):
            for t in range(D // 16):
                sl = pl.ds(t * 16, 16)
                kk = krows[r, sl]
                qq = qrows[r, sl]
                vv = vrows[r, sl]
                gate = 1.0 / (1.0 + jnp.exp(-(kk + qq)))
                krows[r, sl] = gate * vv
            return 0

        lax.fori_loop(0, B, row, 0)
        pltpu.sync_copy(krows, aggr.at[dstv], add=True)
        return 0

    lax.fori_loop(0, NCHUNK, chunk, 0)
    plsc.subcore_barrier()

    r0 = sid * RPS
    pltpu.sync_copy(aggr.at[pl.ds(r0, RPS)],
                    out_hbm.at[pl.ds(cid * NPAD + r0, RPS)])


def kernel(x, edge_index, W_pre, b_pre, g_pre, be_pre, Wk, Wq, Wv, Ws,
           bk, bq, bv, bs, gamma, beta, Wh1, bh1, Wh2, bh2):
    src = edge_index[0]
    dst = edge_index[1]
    r = lambda t: t.reshape(1, -1)

    h, k, q, v, s = _pre_call(
        x, W_pre, r(b_pre), r(g_pre), r(be_pre),
        Wk[0], Wq[0], Wv[0], Ws[0], r(bk[0]), r(bq[0]), r(bv[0]), r(bs[0]))

    for l in range(L):
        aggr2 = _edge_call(k, q, v, src, dst)
        if l < L - 1:
            h, k, q, v, s = _mid_call(
                aggr2, h, s, r(gamma[l]), r(beta[l]),
                Wk[l + 1], Wq[l + 1], Wv[l + 1], Ws[l + 1],
                r(bk[l + 1]), r(bq[l + 1]), r(bv[l + 1]), r(bs[l + 1]))
        else:
            pred = _final_call(
                aggr2, h, s, r(gamma[l]), r(beta[l]),
                Wh1, r(bh1), Wh2, r(bh2))
    return pred
